# baseline (speedup 1.0000x reference)
import functools

import jax
import jax.numpy as jnp
import numpy as np
from jax.experimental import pallas as pl
from jax.experimental.pallas import tpu as pltpu

_NUM_CLASSES = 51
_PAD_CLASSES = 64
_FEAT = 1024
_HEAD = (0, 31, 20, 48, 30, 22, 29, 8, 50, 21)


def _class_weights() -> np.ndarray:
    w = np.zeros((_PAD_CLASSES, 8), dtype=np.float32)
    head = set(_HEAD)
    for k in range(_NUM_CLASSES):
        fg = 1.0 if k != 0 else 0.0
        hm = 1.0 if k in head else 0.0
        w[k, 0] = fg
        w[k, 1] = fg * hm
        w[k, 2] = fg * (1.0 - hm)
    w[:, 3] = np.arange(_PAD_CLASSES, dtype=np.float32)
    return w


def _body(x_ref, lbl_ref, cen_ref, w_ref, out_ref):
    xb = x_ref[...]
    lbl = lbl_ref[0]
    bsz = xb.shape[0]

    cls = jax.lax.broadcasted_iota(jnp.int32, (_PAD_CLASSES, bsz), 0)
    oh = (cls == lbl).astype(jnp.float32)

    cnorm = jnp.sum(cen_ref[...] * cen_ref[...], axis=1, keepdims=True)
    wlane = jax.lax.broadcasted_iota(jnp.int32, (_PAD_CLASSES, 8), 1)
    wall = jnp.where(wlane == 7, cnorm, w_ref[...])
    cols = jax.lax.dot_general(
        oh, wall,
        dimension_numbers=(((0,), (0,)), ((), ())),
        preferred_element_type=jnp.float32,
        precision=jax.lax.Precision.DEFAULT,
    )
    w_fg = cols[:, 0:1]
    w_h = cols[:, 1:2]
    w_nh = cols[:, 2:3]
    lblf = cols[:, 3:4]
    csq = cols[:, 7:8]

    p = jax.lax.dot_general(
        xb, cen_ref[...],
        dimension_numbers=(((1,), (1,)), ((), ())),
        preferred_element_type=jnp.float32,
        precision=jax.lax.Precision.DEFAULT,
    )
    ones = jnp.ones((_FEAT, 8), jnp.float32)
    xsq = jax.lax.dot_general(
        xb * xb, ones,
        dimension_numbers=(((1,), (0,)), ((), ())),
        preferred_element_type=jnp.float32,
        precision=jax.lax.Precision.DEFAULT,
    )[:, 0:1]

    cls2 = jax.lax.broadcasted_iota(jnp.int32, (bsz, _PAD_CLASSES), 1)
    oht = (cls2 == lblf.astype(jnp.int32)).astype(jnp.float32)
    xc = jnp.sum(p * oht, axis=1, keepdims=True)

    d = xsq + csq - 2.0 * xc
    cd = jnp.clip(d, 1e-8, 1e8)

    s1 = jnp.sum(w_fg * cd)
    s2 = jnp.sum(w_h * cd + w_nh * 1e-8)
    s3 = jnp.sum(w_fg)

    rows = jax.lax.broadcasted_iota(jnp.int32, (8, 128), 0)
    blk = jnp.where(rows == 0, s1, jnp.where(rows == 1, s2,
                    jnp.where(rows == 2, s3, 0.0)))
    out_ref[0] = blk


@functools.partial(jax.jit, static_argnames=("block",))
def _run(x, cut_labels, centers, block=512):
    n = x.shape[0]
    nb = n // block
    cen = jnp.zeros((_PAD_CLASSES, _FEAT), jnp.float32).at[:_NUM_CLASSES].set(centers)
    lbl = cut_labels.astype(jnp.int32).reshape(nb, 1, block)
    w = jnp.asarray(_class_weights())

    out = pl.pallas_call(
        _body,
        grid=(nb,),
        in_specs=[
            pl.BlockSpec((block, _FEAT), lambda i: (i, 0)),
            pl.BlockSpec((1, 1, block), lambda i: (i, 0, 0)),
            pl.BlockSpec((_PAD_CLASSES, _FEAT), lambda i: (0, 0)),
            pl.BlockSpec((_PAD_CLASSES, 8), lambda i: (0, 0)),
        ],
        out_specs=pl.BlockSpec((1, 8, 128), lambda i: (i, 0, 0)),
        out_shape=jax.ShapeDtypeStruct((nb, 8, 128), jnp.float32),
        compiler_params=pltpu.CompilerParams(
            dimension_semantics=("parallel",),
        ),
    )(x, lbl, cen, w)

    s1 = jnp.sum(out[:, 0, 0])
    s2 = jnp.sum(out[:, 1, 0])
    cnt = jnp.maximum(jnp.sum(out[:, 2, 0]), 1.0)
    r1 = s1 / cnt
    r2 = s2 / cnt
    r1 = jnp.where(jnp.isnan(r1), 0.0, r1)
    r2 = jnp.where(jnp.isnan(r2), 0.0, r2)
    return r1, r2


def kernel(x, cut_labels, logits, labels, centers):
    del logits, labels
    return _run(x, cut_labels, centers)


# speedup vs baseline: 1.0373x; 1.0373x over previous
import functools

import jax
import jax.numpy as jnp
import numpy as np
from jax.experimental import pallas as pl
from jax.experimental.pallas import tpu as pltpu

_NUM_CLASSES = 51
_PAD_CLASSES = 64
_FEAT = 1024
_HEAD = (0, 31, 20, 48, 30, 22, 29, 8, 50, 21)


def _class_weights() -> np.ndarray:
    w = np.zeros((_PAD_CLASSES, 8), dtype=np.float32)
    head = set(_HEAD)
    for k in range(_NUM_CLASSES):
        fg = 1.0 if k != 0 else 0.0
        hm = 1.0 if k in head else 0.0
        w[k, 0] = fg
        w[k, 1] = fg * hm
        w[k, 2] = fg * (1.0 - hm)
    w[:, 3] = np.arange(_PAD_CLASSES, dtype=np.float32)
    return w


def _body(x_ref, lbl_ref, cen_ref, w_ref, out_ref):
    xb = x_ref[...]
    lbl = lbl_ref[0]
    bsz = xb.shape[0]

    cls = jax.lax.broadcasted_iota(jnp.int32, (_PAD_CLASSES, bsz), 0)
    oh = (cls == lbl).astype(jnp.float32)

    cnorm = jnp.sum(cen_ref[...] * cen_ref[...], axis=1, keepdims=True)
    wlane = jax.lax.broadcasted_iota(jnp.int32, (_PAD_CLASSES, 8), 1)
    wall = jnp.where(wlane == 7, cnorm, w_ref[...])
    cols = jax.lax.dot_general(
        oh, wall,
        dimension_numbers=(((0,), (0,)), ((), ())),
        preferred_element_type=jnp.float32,
        precision=jax.lax.Precision.DEFAULT,
    )
    w_fg = cols[:, 0:1]
    w_h = cols[:, 1:2]
    w_nh = cols[:, 2:3]
    lblf = cols[:, 3:4]
    csq = cols[:, 7:8]

    p = jax.lax.dot_general(
        xb, cen_ref[...],
        dimension_numbers=(((1,), (1,)), ((), ())),
        preferred_element_type=jnp.float32,
        precision=jax.lax.Precision.DEFAULT,
    )
    ones = jnp.ones((_FEAT, 8), jnp.float32)
    xsq = jax.lax.dot_general(
        xb * xb, ones,
        dimension_numbers=(((1,), (0,)), ((), ())),
        preferred_element_type=jnp.float32,
        precision=jax.lax.Precision.DEFAULT,
    )[:, 0:1]

    cls2 = jax.lax.broadcasted_iota(jnp.int32, (bsz, _PAD_CLASSES), 1)
    oht = (cls2 == lblf.astype(jnp.int32)).astype(jnp.float32)
    xc = jnp.sum(p * oht, axis=1, keepdims=True)

    d = xsq + csq - 2.0 * xc
    cd = jnp.clip(d, 1e-8, 1e8)

    s1 = jnp.sum(w_fg * cd)
    s2 = jnp.sum(w_h * cd + w_nh * 1e-8)
    s3 = jnp.sum(w_fg)

    rows = jax.lax.broadcasted_iota(jnp.int32, (8, 128), 0)
    blk = jnp.where(rows == 0, s1, jnp.where(rows == 1, s2,
                    jnp.where(rows == 2, s3, 0.0)))
    out_ref[0] = blk


@functools.partial(jax.jit, static_argnames=("block",))
def _run(x, cut_labels, centers, block=2048):
    n = x.shape[0]
    nb = n // block
    cen = jnp.zeros((_PAD_CLASSES, _FEAT), jnp.float32).at[:_NUM_CLASSES].set(centers)
    lbl = cut_labels.astype(jnp.int32).reshape(nb, 1, block)
    w = jnp.asarray(_class_weights())

    out = pl.pallas_call(
        _body,
        grid=(nb,),
        in_specs=[
            pl.BlockSpec((block, _FEAT), lambda i: (i, 0)),
            pl.BlockSpec((1, 1, block), lambda i: (i, 0, 0)),
            pl.BlockSpec((_PAD_CLASSES, _FEAT), lambda i: (0, 0)),
            pl.BlockSpec((_PAD_CLASSES, 8), lambda i: (0, 0)),
        ],
        out_specs=pl.BlockSpec((1, 8, 128), lambda i: (i, 0, 0)),
        out_shape=jax.ShapeDtypeStruct((nb, 8, 128), jnp.float32),
        compiler_params=pltpu.CompilerParams(
            dimension_semantics=("arbitrary",),
        ),
    )(x, lbl, cen, w)

    s1 = jnp.sum(out[:, 0, 0])
    s2 = jnp.sum(out[:, 1, 0])
    cnt = jnp.maximum(jnp.sum(out[:, 2, 0]), 1.0)
    r1 = s1 / cnt
    r2 = s2 / cnt
    r1 = jnp.where(jnp.isnan(r1), 0.0, r1)
    r2 = jnp.where(jnp.isnan(r2), 0.0, r2)
    return r1, r2


def kernel(x, cut_labels, logits, labels, centers):
    del logits, labels
    return _run(x, cut_labels, centers)
